# Initial kernel scaffold; baseline (speedup 1.0000x reference)
import jax
import jax.numpy as jnp
from jax.experimental import pallas as pl


def kernel(x):
    raise NotImplementedError("write your pallas kernel here")


# speedup vs baseline: 1.0698x; 1.0698x over previous
import functools

import numpy as np

import jax
import jax.numpy as jnp
from jax import lax
from jax.experimental import pallas as pl
from jax.experimental.pallas import tpu as pltpu
from jax.experimental.pallas import tpu_sc as plsc

B = 1024
F = 26
D = 128
LANES = 16
NCH = D // LANES
NUM_PAIRS = F * (F - 1) // 2
PW = NUM_PAIRS * LANES
NUM_WORKERS = 32
BPW = B // NUM_WORKERS
CHUNK = 8
NCHUNKS = BPW // CHUNK
JBLK = 5


def _pair_index(i, j):
    return i * (2 * F - i - 1) // 2 + (j - i - 1)


_MESH = plsc.VectorSubcoreMesh(core_axis_name="c", subcore_axis_name="s")


@functools.partial(
    pl.kernel,
    mesh=_MESH,
    out_type=jax.ShapeDtypeStruct((B, PW), jnp.float32),
    scratch_types=[
        pltpu.VMEM((CHUNK, F * D), jnp.float32),

        pltpu.VMEM((CHUNK, PW), jnp.float32),
    ],
)
def _ipn_sc_partial(x_hbm, part_hbm, xv, sv):
    wid = lax.axis_index("s") * 2 + lax.axis_index("c")
    base = wid * BPW

    def chunk_body(c, carry):
        pltpu.sync_copy(x_hbm.at[pl.ds(base + c * CHUNK, CHUNK)], xv)

        def batch_body(b, carry2):
            for j0 in range(1, F, JBLK):
                js = list(range(j0, min(j0 + JBLK, F)))
                cache = {
                    j: [
                        xv[b, pl.ds(j * D + ch * LANES, LANES)]
                        for ch in range(NCH)
                    ]
                    for j in js
                }
                for i in range(js[-1]):
                    xi = [
                        xv[b, pl.ds(i * D + ch * LANES, LANES)]
                        for ch in range(NCH)
                    ]
                    for j in js:
                        if j <= i:
                            continue
                        acc = xi[0] * cache[j][0]
                        for ch in range(1, NCH):
                            acc = acc + xi[ch] * cache[j][ch]
                        sv[b, pl.ds(_pair_index(i, j) * LANES, LANES)] = acc
            return carry2

        lax.fori_loop(0, CHUNK, batch_body, 0)
        pltpu.sync_copy(sv, part_hbm.at[pl.ds(base + c * CHUNK, CHUNK)])
        return carry

    lax.fori_loop(0, NCHUNKS, chunk_body, 0)


def _fold_body(p_ref, g_ref, o_ref):
    t = p_ref[...].astype(jnp.bfloat16)
    o_ref[...] = jnp.dot(t, g_ref[...], preferred_element_type=jnp.float32)


_G_NP = np.zeros((PW, NUM_PAIRS), np.float32)
for _p in range(NUM_PAIRS):
    _G_NP[_p * LANES : (_p + 1) * LANES, _p] = 1.0

_BB = 128


def _fold_tc(part):
    return pl.pallas_call(
        _fold_body,
        grid=(B // _BB,),
        in_specs=[
            pl.BlockSpec((_BB, PW), lambda i: (i, 0)),
            pl.BlockSpec((PW, NUM_PAIRS), lambda i: (0, 0)),
        ],
        out_specs=pl.BlockSpec((_BB, NUM_PAIRS), lambda i: (i, 0)),
        out_shape=jax.ShapeDtypeStruct((B, NUM_PAIRS), jnp.float32),
    )(part, jnp.asarray(_G_NP, jnp.bfloat16))


def kernel(x):
    part = _ipn_sc_partial(x.reshape(B, F * D))
    return _fold_tc(part)


# speedup vs baseline: 1.1345x; 1.0605x over previous
import functools

import numpy as np

import jax
import jax.numpy as jnp
from jax import lax
from jax.experimental import pallas as pl
from jax.experimental.pallas import tpu as pltpu
from jax.experimental.pallas import tpu_sc as plsc

B = 1024
F = 26
D = 128
LANES = 16
NCH = D // LANES
NUM_PAIRS = F * (F - 1) // 2
PW = NUM_PAIRS * LANES
NUM_WORKERS = 32
BPW = B // NUM_WORKERS
CHUNK = 4
NCHUNKS = BPW // CHUNK
JBLK = 5


def _pair_index(i, j):
    return i * (2 * F - i - 1) // 2 + (j - i - 1)


_MESH = plsc.VectorSubcoreMesh(core_axis_name="c", subcore_axis_name="s")


@functools.partial(
    pl.kernel,
    mesh=_MESH,
    out_type=jax.ShapeDtypeStruct((B, PW), jnp.float32),
    scratch_types=[
        pltpu.VMEM((2 * CHUNK, F, D), jnp.float32),
        pltpu.VMEM((2 * CHUNK, PW), jnp.float32),
        pltpu.SemaphoreType.DMA,
        pltpu.SemaphoreType.DMA,
    ],
)
def _ipn_sc_partial(x_hbm, part_hbm, xv, sv, sem_in, sem_out):
    wid = lax.axis_index("s") * 2 + lax.axis_index("c")
    base = wid * BPW

    def in_cp(t, half):
        off = jnp.minimum(base + t * CHUNK, B - CHUNK)
        return pltpu.make_async_copy(
            x_hbm.at[pl.ds(off, CHUNK)],
            xv.at[pl.ds(half * CHUNK, CHUNK)],
            sem_in,
        )

    def out_cp(t, half):
        return pltpu.make_async_copy(
            sv.at[pl.ds(half * CHUNK, CHUNK)],
            part_hbm.at[pl.ds(base + t * CHUNK, CHUNK)],
            sem_out,
        )

    in_cp(0, 0).start()
    out_cp(NCHUNKS - 1, 1).start()

    def chunk_body(t, carry):
        par = jnp.bitwise_and(t, 1)
        in_cp(t, par).wait()
        in_cp(t + 1, 1 - par).start()

        def batch_body(b, carry2):
            bb = par * CHUNK + b
            for j0 in range(1, F, JBLK):
                js = list(range(j0, min(j0 + JBLK, F)))
                cache = {
                    j: [
                        xv[bb, j, pl.ds(ch * LANES, LANES)]
                        for ch in range(NCH)
                    ]
                    for j in js
                }
                for i in range(js[-1]):
                    xi = [
                        xv[bb, i, pl.ds(ch * LANES, LANES)]
                        for ch in range(NCH)
                    ]
                    for j in js:
                        if j <= i:
                            continue
                        acc = xi[0] * cache[j][0]
                        for ch in range(1, NCH):
                            acc = acc + xi[ch] * cache[j][ch]
                        sv[bb, pl.ds(_pair_index(i, j) * LANES, LANES)] = acc
            return carry2

        lax.fori_loop(0, CHUNK, batch_body, 0)
        out_cp(t, par).wait()
        out_cp(t, par).start()
        return carry

    lax.fori_loop(0, NCHUNKS, chunk_body, 0)
    in_cp(NCHUNKS, NCHUNKS % 2).wait()
    out_cp(NCHUNKS - 1, jnp.int32(1 - (NCHUNKS - 1) % 2)).wait()


def _fold_body(p_ref, g_ref, o_ref):
    t = p_ref[...].astype(jnp.bfloat16)
    o_ref[...] = jnp.dot(t, g_ref[...], preferred_element_type=jnp.float32)


_G_NP = np.zeros((PW, NUM_PAIRS), np.float32)
for _p in range(NUM_PAIRS):
    _G_NP[_p * LANES : (_p + 1) * LANES, _p] = 1.0

_BB = 128


def _fold_tc(part):
    return pl.pallas_call(
        _fold_body,
        grid=(B // _BB,),
        in_specs=[
            pl.BlockSpec((_BB, PW), lambda i: (i, 0)),
            pl.BlockSpec((PW, NUM_PAIRS), lambda i: (0, 0)),
        ],
        out_specs=pl.BlockSpec((_BB, NUM_PAIRS), lambda i: (i, 0)),
        out_shape=jax.ShapeDtypeStruct((B, NUM_PAIRS), jnp.float32),
    )(part, jnp.asarray(_G_NP, jnp.bfloat16))


def kernel(x):
    part = _ipn_sc_partial(x)
    return _fold_tc(part)
